# Initial kernel scaffold; baseline (speedup 1.0000x reference)
import jax
import jax.numpy as jnp
from jax.experimental import pallas as pl


def kernel(atom_positions, bond_atom_indices, pbc_offsets, lattices):
    raise NotImplementedError("write your pallas kernel here")


# speedup vs baseline: 1.9286x; 1.9286x over previous
import functools

import jax
import jax.numpy as jnp
from jax import lax
from jax.experimental import pallas as pl
from jax.experimental.pallas import tpu as pltpu
from jax.experimental.pallas import tpu_sc as plsc

N_NODES_K = 100000
N_EDGES_K = 3200000

N_TILES = 32
B = 1024
IDX_SLICE = 128
N_SLICES = (2 * B) // IDX_SLICE
N_CHUNKS = N_EDGES_K // B
BASE_CHUNKS = N_CHUNKS // N_TILES
EXTRA = N_CHUNKS - BASE_CHUNKS * N_TILES

_LANES = 16


def _rsqrt_newton(x):
    i = lax.bitcast_convert_type(x, jnp.int32)
    i = jnp.int32(0x5F3759DF) - lax.shift_right_arithmetic(i, jnp.int32(1))
    y = lax.bitcast_convert_type(i, jnp.float32)
    for _ in range(3):
        y = y * (1.5 - 0.5 * x * y * y)
    return y


def _body(pos_hbm, bond_hbm, pbc_hbm, lat_hbm, out_hbm,
          idx_v, gath_v, pbc_v, out_v, lat_v, gsem):
    wid = lax.axis_index("s") * 2 + lax.axis_index("c")
    pltpu.sync_copy(lat_hbm, lat_v)
    lane0 = lax.iota(jnp.int32, _LANES)
    lat = [lat_v[k] for k in range(9)]
    nchunks = jnp.int32(BASE_CHUNKS) + jnp.where(wid < EXTRA, 1, 0).astype(jnp.int32)

    def chunk_body(i, carry):
        cid = wid + i * N_TILES
        ebase = cid * B
        pltpu.sync_copy(bond_hbm.at[pl.ds(2 * ebase, 2 * B)], idx_v)
        copies = []
        for k in range(N_SLICES):
            copies.append(pltpu.async_copy(
                pos_hbm.at[idx_v.at[pl.ds(k * IDX_SLICE, IDX_SLICE)]],
                gath_v.at[pl.ds(k * IDX_SLICE, IDX_SLICE)], gsem))
        pltpu.sync_copy(pbc_hbm.at[pl.ds(3 * ebase, 3 * B)], pbc_v)
        for c in copies:
            c.wait()

        def step(s, c2):
            ids = lane0 + s * _LANES
            ri = ids * 2
            rj = ri + 1
            c0 = jnp.full((_LANES,), 0, jnp.int32)
            c1 = jnp.full((_LANES,), 1, jnp.int32)
            c2c = jnp.full((_LANES,), 2, jnp.int32)
            xi = plsc.load_gather(gath_v, [ri, c0])
            yi = plsc.load_gather(gath_v, [ri, c1])
            zi = plsc.load_gather(gath_v, [ri, c2c])
            xj = plsc.load_gather(gath_v, [rj, c0])
            yj = plsc.load_gather(gath_v, [rj, c1])
            zj = plsc.load_gather(gath_v, [rj, c2c])
            p = ids * 3
            ox = plsc.load_gather(pbc_v, [p])
            oy = plsc.load_gather(pbc_v, [p + 1])
            oz = plsc.load_gather(pbc_v, [p + 2])
            dx = xj - xi + (ox * lat[0] + oy * lat[3] + oz * lat[6])
            dy = yj - yi + (ox * lat[1] + oy * lat[4] + oz * lat[7])
            dz = zj - zi + (ox * lat[2] + oy * lat[5] + oz * lat[8])
            ss = dx * dx + dy * dy + dz * dz
            ss = jnp.maximum(ss, jnp.float32(1e-30))
            out_v[pl.ds(s * _LANES, _LANES)] = ss * _rsqrt_newton(ss)
            return c2

        lax.fori_loop(0, B // _LANES, step, jnp.int32(0), unroll=2)
        pltpu.sync_copy(out_v, out_hbm.at[pl.ds(ebase, B)])
        return carry

    lax.fori_loop(0, nchunks, chunk_body, jnp.int32(0))


_sc_call = pl.kernel(
    _body,
    out_type=jax.ShapeDtypeStruct((N_EDGES_K,), jnp.float32),
    mesh=plsc.VectorSubcoreMesh(core_axis_name="c", subcore_axis_name="s", num_cores=2, num_subcores=16),
    scratch_types=[
        pltpu.VMEM((2 * B,), jnp.int32),
        pltpu.VMEM((2 * B, 8), jnp.float32),
        pltpu.VMEM((3 * B,), jnp.float32),
        pltpu.VMEM((B,), jnp.float32),
        pltpu.VMEM((9, _LANES), jnp.float32),
        pltpu.SemaphoreType.DMA,
    ],
    compiler_params=pltpu.CompilerParams(
        needs_layout_passes=False, use_tc_tiling_on_sc=False),
)


def kernel(atom_positions, bond_atom_indices, pbc_offsets, lattices):
    pos8 = jnp.pad(atom_positions, ((0, 0), (0, 5)))
    bond_flat = bond_atom_indices.reshape(-1)
    pbc_flat = pbc_offsets.reshape(-1)
    lat_b = jnp.broadcast_to(lattices.reshape(9, 1), (9, 16))
    return _sc_call(pos8, bond_flat, pbc_flat, lat_b)


# speedup vs baseline: 34.5613x; 17.9207x over previous
import jax
import jax.numpy as jnp
from jax import lax
from jax.experimental import pallas as pl
from jax.experimental.pallas import tpu as pltpu
from jax.experimental.pallas import tpu_sc as plsc

_N_NODES = 100000
_N_EDGES = 3200000

N_TILES = 32
B = 1024
BLK = 128
N_BLK = B // BLK
N_CHUNKS = _N_EDGES // B
BASE_CHUNKS = N_CHUNKS // N_TILES
EXTRA = N_CHUNKS - BASE_CHUNKS * N_TILES

_LANES = 16


def _rsqrt_newton(x):
    i = lax.bitcast_convert_type(x, jnp.int32)
    i = jnp.int32(0x5F3759DF) - lax.shift_right_arithmetic(i, jnp.int32(1))
    y = lax.bitcast_convert_type(i, jnp.float32)
    for _ in range(3):
        y = y * (1.5 - 0.5 * x * y * y)
    return y


def _body(pos_hbm, bond_hbm, pbc_hbm, lat_hbm, out_hbm,
          idx_v, gi_v, gj_v, px_v, py_v, pz_v, out_v, lat_v, gsem):
    wid = lax.axis_index("s") * 2 + lax.axis_index("c")
    pltpu.sync_copy(lat_hbm, lat_v)
    lane0 = lax.iota(jnp.int32, _LANES)
    lat = [lat_v[k] for k in range(9)]
    nchunks = jnp.int32(BASE_CHUNKS) + jnp.where(wid < EXTRA, 1, 0).astype(jnp.int32)

    def chunk_body(i, carry):
        cid = wid + i * N_TILES
        ebase = cid * B
        pltpu.sync_copy(bond_hbm.at[pl.ds(cid * N_BLK, N_BLK)], idx_v)
        copies = []
        for b in range(N_BLK):
            copies.append(pltpu.async_copy(
                pos_hbm.at[idx_v.at[b, 0]],
                gi_v.at[pl.ds(b * BLK, BLK)], gsem))
            copies.append(pltpu.async_copy(
                pos_hbm.at[idx_v.at[b, 1]],
                gj_v.at[pl.ds(b * BLK, BLK)], gsem))
        pltpu.sync_copy(pbc_hbm.at[pl.ds(ebase, B)], px_v)
        pltpu.sync_copy(pbc_hbm.at[pl.ds(_N_EDGES + ebase, B)], py_v)
        pltpu.sync_copy(pbc_hbm.at[pl.ds(2 * _N_EDGES + ebase, B)], pz_v)
        for c in copies:
            c.wait()

        def step(s, c2):
            ids = lane0 + s * _LANES
            c0 = jnp.full((_LANES,), 0, jnp.int32)
            c1 = jnp.full((_LANES,), 1, jnp.int32)
            c2c = jnp.full((_LANES,), 2, jnp.int32)
            xi = plsc.load_gather(gi_v, [ids, c0])
            yi = plsc.load_gather(gi_v, [ids, c1])
            zi = plsc.load_gather(gi_v, [ids, c2c])
            xj = plsc.load_gather(gj_v, [ids, c0])
            yj = plsc.load_gather(gj_v, [ids, c1])
            zj = plsc.load_gather(gj_v, [ids, c2c])
            sl = pl.ds(s * _LANES, _LANES)
            ox = px_v[sl]
            oy = py_v[sl]
            oz = pz_v[sl]
            dx = xj - xi + (ox * lat[0] + oy * lat[3] + oz * lat[6])
            dy = yj - yi + (ox * lat[1] + oy * lat[4] + oz * lat[7])
            dz = zj - zi + (ox * lat[2] + oy * lat[5] + oz * lat[8])
            ss = dx * dx + dy * dy + dz * dz
            ss = jnp.maximum(ss, jnp.float32(1e-30))
            out_v[sl] = ss * _rsqrt_newton(ss)
            return c2

        lax.fori_loop(0, B // _LANES, step, jnp.int32(0), unroll=2)
        pltpu.sync_copy(out_v, out_hbm.at[pl.ds(ebase, B)])
        return carry

    lax.fori_loop(0, nchunks, chunk_body, jnp.int32(0))


_sc_call = pl.kernel(
    _body,
    out_type=jax.ShapeDtypeStruct((_N_EDGES,), jnp.float32),
    mesh=plsc.VectorSubcoreMesh(core_axis_name="c", subcore_axis_name="s",
                                num_cores=2, num_subcores=16),
    scratch_types=[
        pltpu.VMEM((N_BLK, 2, BLK), jnp.int32),
        pltpu.VMEM((B, 8), jnp.float32),
        pltpu.VMEM((B, 8), jnp.float32),
        pltpu.VMEM((B,), jnp.float32),
        pltpu.VMEM((B,), jnp.float32),
        pltpu.VMEM((B,), jnp.float32),
        pltpu.VMEM((B,), jnp.float32),
        pltpu.VMEM((9, _LANES), jnp.float32),
        pltpu.SemaphoreType.DMA,
    ],
    compiler_params=pltpu.CompilerParams(
        needs_layout_passes=False, use_tc_tiling_on_sc=False),
)


def kernel(atom_positions, bond_atom_indices, pbc_offsets, lattices):
    pos8 = jnp.pad(atom_positions, ((0, 0), (0, 5)))
    bond_blocks = jnp.transpose(
        bond_atom_indices.reshape(_N_EDGES // BLK, BLK, 2), (0, 2, 1))
    pbc_t = jnp.transpose(pbc_offsets)
    pbc_soa = jnp.concatenate([pbc_t[0], pbc_t[1], pbc_t[2]])
    lat_b = jnp.broadcast_to(lattices.reshape(9, 1), (9, 16))
    return _sc_call(pos8, bond_blocks, pbc_soa, lat_b)


# speedup vs baseline: 47.1009x; 1.3628x over previous
import jax
import jax.numpy as jnp
from jax import lax
from jax.experimental import pallas as pl
from jax.experimental.pallas import tpu as pltpu
from jax.experimental.pallas import tpu_sc as plsc

_N_NODES = 100000
_N_EDGES = 3200000

N_TILES = 32
B = 1024
BLK = 128
N_BLK = B // BLK
N_CHUNKS = _N_EDGES // B
BASE_CHUNKS = N_CHUNKS // N_TILES
EXTRA = N_CHUNKS - BASE_CHUNKS * N_TILES

_LANES = 16


def _rsqrt_newton(x):
    i = lax.bitcast_convert_type(x, jnp.int32)
    i = jnp.int32(0x5F3759DF) - lax.shift_right_arithmetic(i, jnp.int32(1))
    y = lax.bitcast_convert_type(i, jnp.float32)
    for _ in range(2):
        y = y * (1.5 - 0.5 * x * y * y)
    return y


def _body(pos_hbm, bond_hbm, pbc_hbm, lat_hbm, out_hbm,
          idx_v, gi_v, gj_v, px_v, py_v, pz_v, out_v, lat_v, gsem, osem):
    wid = lax.axis_index("s") * 2 + lax.axis_index("c")
    pltpu.sync_copy(lat_hbm, lat_v)
    lane0 = lax.iota(jnp.int32, _LANES)
    lat = [lat_v[k] for k in range(9)]
    nchunks = jnp.int32(BASE_CHUNKS) + jnp.where(wid < EXTRA, 1, 0).astype(jnp.int32)

    def _gather_descs(p):
        descs = []
        for b in range(N_BLK):
            descs.append(pltpu.make_async_copy(
                pos_hbm.at[idx_v.at[p, b, 0]],
                gi_v.at[p].at[pl.ds(b * BLK, BLK)], gsem))
            descs.append(pltpu.make_async_copy(
                pos_hbm.at[idx_v.at[p, b, 1]],
                gj_v.at[p].at[pl.ds(b * BLK, BLK)], gsem))
        return descs

    def _pbc_descs(cid, p):
        ebase = cid * B
        return [
            pltpu.make_async_copy(
                pbc_hbm.at[pl.ds(ebase, B)], px_v.at[p], gsem),
            pltpu.make_async_copy(
                pbc_hbm.at[pl.ds(_N_EDGES + ebase, B)], py_v.at[p], gsem),
            pltpu.make_async_copy(
                pbc_hbm.at[pl.ds(2 * _N_EDGES + ebase, B)], pz_v.at[p], gsem),
        ]

    def load_and_start(cid, p):
        pltpu.sync_copy(bond_hbm.at[pl.ds(cid * N_BLK, N_BLK)], idx_v.at[p])
        for d in _gather_descs(p) + _pbc_descs(cid, p):
            d.start()

    def wait_chunk(cid, p):
        for d in _gather_descs(p) + _pbc_descs(cid, p):
            d.wait()

    def out_desc(cid, p):
        return pltpu.make_async_copy(
            out_v.at[p], out_hbm.at[pl.ds(cid * B, B)], osem)

    def compute(cid, p):
        gi = gi_v.at[p]
        gj = gj_v.at[p]
        px = px_v.at[p]
        py = py_v.at[p]
        pz = pz_v.at[p]
        ov = out_v.at[p]

        def step(s, c2):
            ids = lane0 + s * _LANES
            c0 = jnp.full((_LANES,), 0, jnp.int32)
            c1 = jnp.full((_LANES,), 1, jnp.int32)
            c2c = jnp.full((_LANES,), 2, jnp.int32)
            xi = plsc.load_gather(gi, [ids, c0])
            yi = plsc.load_gather(gi, [ids, c1])
            zi = plsc.load_gather(gi, [ids, c2c])
            xj = plsc.load_gather(gj, [ids, c0])
            yj = plsc.load_gather(gj, [ids, c1])
            zj = plsc.load_gather(gj, [ids, c2c])
            sl = pl.ds(s * _LANES, _LANES)
            ox = px[sl]
            oy = py[sl]
            oz = pz[sl]
            dx = xj - xi + (ox * lat[0] + oy * lat[3] + oz * lat[6])
            dy = yj - yi + (ox * lat[1] + oy * lat[4] + oz * lat[7])
            dz = zj - zi + (ox * lat[2] + oy * lat[5] + oz * lat[8])
            ss = dx * dx + dy * dy + dz * dz
            ss = jnp.maximum(ss, jnp.float32(1e-30))
            ov[sl] = ss * _rsqrt_newton(ss)
            return c2

        lax.fori_loop(0, B // _LANES, step, jnp.int32(0), unroll=2)
        out_desc(cid, p).start()

    load_and_start(wid, 0)
    npairs = (nchunks + 1) // 2

    def pair_body(k, carry):
        c_a = wid + (2 * k) * N_TILES
        c_b = c_a + N_TILES
        has_b = (2 * k + 1) < nchunks

        wait_chunk(c_a, 0)

        @pl.when(has_b)
        def _():
            load_and_start(c_b, 1)

        @pl.when(k > 0)
        def _():
            out_desc(c_a, 0).wait()

        compute(c_a, 0)

        @pl.when(has_b)
        def _():
            wait_chunk(c_b, 1)

            @pl.when((2 * k + 2) < nchunks)
            def _():
                load_and_start(wid + (2 * k + 2) * N_TILES, 0)

            @pl.when(k > 0)
            def _():
                out_desc(c_b, 1).wait()

            compute(c_b, 1)

        return carry

    lax.fori_loop(0, npairs, pair_body, jnp.int32(0))
    out_desc(wid, 0).wait()
    out_desc(wid, 1).wait()


_sc_call = pl.kernel(
    _body,
    out_type=jax.ShapeDtypeStruct((_N_EDGES,), jnp.float32),
    mesh=plsc.VectorSubcoreMesh(core_axis_name="c", subcore_axis_name="s",
                                num_cores=2, num_subcores=16),
    scratch_types=[
        pltpu.VMEM((2, N_BLK, 2, BLK), jnp.int32),
        pltpu.VMEM((2, B, 8), jnp.float32),
        pltpu.VMEM((2, B, 8), jnp.float32),
        pltpu.VMEM((2, B), jnp.float32),
        pltpu.VMEM((2, B), jnp.float32),
        pltpu.VMEM((2, B), jnp.float32),
        pltpu.VMEM((2, B), jnp.float32),
        pltpu.VMEM((9, _LANES), jnp.float32),
        pltpu.SemaphoreType.DMA,
        pltpu.SemaphoreType.DMA,
    ],
    compiler_params=pltpu.CompilerParams(
        needs_layout_passes=False, use_tc_tiling_on_sc=False),
)


def kernel(atom_positions, bond_atom_indices, pbc_offsets, lattices):
    pos8 = jnp.pad(atom_positions, ((0, 0), (0, 5)))
    bond_blocks = jnp.transpose(
        bond_atom_indices.reshape(_N_EDGES // BLK, BLK, 2), (0, 2, 1))
    pbc_t = jnp.transpose(pbc_offsets)
    pbc_soa = jnp.concatenate([pbc_t[0], pbc_t[1], pbc_t[2]])
    lat_b = jnp.broadcast_to(lattices.reshape(9, 1), (9, 16))
    return _sc_call(pos8, bond_blocks, pbc_soa, lat_b)


# speedup vs baseline: 52.2002x; 1.1083x over previous
import jax
import jax.numpy as jnp
from jax import lax
from jax.experimental import pallas as pl
from jax.experimental.pallas import tpu as pltpu
from jax.experimental.pallas import tpu_sc as plsc

_N_NODES = 100000
_N_EDGES = 3200000

N_TILES = 32
B = 1024
BLK = 128
N_BLK = B // BLK
N_CHUNKS = _N_EDGES // B
BASE_CHUNKS = N_CHUNKS // N_TILES
EXTRA = N_CHUNKS - BASE_CHUNKS * N_TILES

_LANES = 16


def _rsqrt_newton(x):
    i = lax.bitcast_convert_type(x, jnp.int32)
    i = jnp.int32(0x5F3759DF) - lax.shift_right_arithmetic(i, jnp.int32(1))
    y = lax.bitcast_convert_type(i, jnp.float32)
    for _ in range(2):
        y = y * (1.5 - 0.5 * x * y * y)
    return y


def _body(pos_hbm, bond_hbm, pbc_hbm, lat_hbm, out_hbm,
          idx_v, gi_v, gj_v, po_v, out_v, lat_v, gsem, osem):
    wid = lax.axis_index("s") * 2 + lax.axis_index("c")
    pltpu.sync_copy(lat_hbm, lat_v)
    lane0 = lax.iota(jnp.int32, _LANES)
    lat = [lat_v[k] for k in range(9)]
    nchunks = jnp.int32(BASE_CHUNKS) + jnp.where(wid < EXTRA, 1, 0).astype(jnp.int32)

    def _gather_descs(p):
        descs = []
        for b in range(N_BLK):
            descs.append(pltpu.make_async_copy(
                pos_hbm.at[idx_v.at[p, b, 0]],
                gi_v.at[p].at[pl.ds(b * BLK, BLK)], gsem))
            descs.append(pltpu.make_async_copy(
                pos_hbm.at[idx_v.at[p, b, 1]],
                gj_v.at[p].at[pl.ds(b * BLK, BLK)], gsem))
        return descs

    def _pbc_descs(cid, p):
        return [
            pltpu.make_async_copy(
                pbc_hbm.at[pl.ds(cid * N_BLK, N_BLK)], po_v.at[p], gsem),
        ]

    def load_and_start(cid, p):
        pltpu.sync_copy(bond_hbm.at[pl.ds(cid * N_BLK, N_BLK)], idx_v.at[p])
        for d in _gather_descs(p) + _pbc_descs(cid, p):
            d.start()

    def wait_chunk(cid, p):
        for d in _gather_descs(p) + _pbc_descs(cid, p):
            d.wait()

    def out_desc(cid, p):
        return pltpu.make_async_copy(
            out_v.at[p], out_hbm.at[pl.ds(cid * B, B)], osem)

    def compute(cid, p):
        gi = gi_v.at[p]
        gj = gj_v.at[p]
        po = po_v.at[p]
        ov = out_v.at[p]

        def step(s, c2):
            ids = lane0 + s * _LANES
            c0 = jnp.full((_LANES,), 0, jnp.int32)
            c1 = jnp.full((_LANES,), 1, jnp.int32)
            c2c = jnp.full((_LANES,), 2, jnp.int32)
            xi = plsc.load_gather(gi, [ids, c0])
            yi = plsc.load_gather(gi, [ids, c1])
            zi = plsc.load_gather(gi, [ids, c2c])
            xj = plsc.load_gather(gj, [ids, c0])
            yj = plsc.load_gather(gj, [ids, c1])
            zj = plsc.load_gather(gj, [ids, c2c])
            blk = s // 8
            off = pl.ds((s % 8) * _LANES, _LANES)
            ox = po[blk, 0, off]
            oy = po[blk, 1, off]
            oz = po[blk, 2, off]
            sl = pl.ds(s * _LANES, _LANES)
            dx = xj - xi + (ox * lat[0] + oy * lat[3] + oz * lat[6])
            dy = yj - yi + (ox * lat[1] + oy * lat[4] + oz * lat[7])
            dz = zj - zi + (ox * lat[2] + oy * lat[5] + oz * lat[8])
            ss = dx * dx + dy * dy + dz * dz
            ss = jnp.maximum(ss, jnp.float32(1e-30))
            ov[sl] = ss * _rsqrt_newton(ss)
            return c2

        lax.fori_loop(0, B // _LANES, step, jnp.int32(0), unroll=2)
        out_desc(cid, p).start()

    load_and_start(wid, 0)
    npairs = (nchunks + 1) // 2

    def pair_body(k, carry):
        c_a = wid + (2 * k) * N_TILES
        c_b = c_a + N_TILES
        has_b = (2 * k + 1) < nchunks

        wait_chunk(c_a, 0)

        @pl.when(has_b)
        def _():
            load_and_start(c_b, 1)

        @pl.when(k > 0)
        def _():
            out_desc(c_a, 0).wait()

        compute(c_a, 0)

        @pl.when(has_b)
        def _():
            wait_chunk(c_b, 1)

            @pl.when((2 * k + 2) < nchunks)
            def _():
                load_and_start(wid + (2 * k + 2) * N_TILES, 0)

            @pl.when(k > 0)
            def _():
                out_desc(c_b, 1).wait()

            compute(c_b, 1)

        return carry

    lax.fori_loop(0, npairs, pair_body, jnp.int32(0))
    out_desc(wid, 0).wait()
    out_desc(wid, 1).wait()


_sc_call = pl.kernel(
    _body,
    out_type=jax.ShapeDtypeStruct((_N_EDGES,), jnp.float32),
    mesh=plsc.VectorSubcoreMesh(core_axis_name="c", subcore_axis_name="s",
                                num_cores=2, num_subcores=16),
    scratch_types=[
        pltpu.VMEM((2, N_BLK, 2, BLK), jnp.int32),
        pltpu.VMEM((2, B, 8), jnp.float32),
        pltpu.VMEM((2, B, 8), jnp.float32),
        pltpu.VMEM((2, N_BLK, 4, BLK), jnp.float32),
        pltpu.VMEM((2, B), jnp.float32),
        pltpu.VMEM((9, _LANES), jnp.float32),
        pltpu.SemaphoreType.DMA,
        pltpu.SemaphoreType.DMA,
    ],
    compiler_params=pltpu.CompilerParams(
        needs_layout_passes=False, use_tc_tiling_on_sc=False),
)


def kernel(atom_positions, bond_atom_indices, pbc_offsets, lattices):
    pos8 = jnp.pad(atom_positions, ((0, 0), (0, 5)))
    bond_blocks = jnp.transpose(
        bond_atom_indices.reshape(_N_EDGES // BLK, BLK, 2), (0, 2, 1))
    pbc_soa = jnp.transpose(
        jnp.pad(pbc_offsets, ((0, 0), (0, 1))).reshape(
            _N_EDGES // BLK, BLK, 4), (0, 2, 1))
    lat_b = jnp.broadcast_to(lattices.reshape(9, 1), (9, 16))
    return _sc_call(pos8, bond_blocks, pbc_soa, lat_b)


# speedup vs baseline: 58.9514x; 1.1293x over previous
import jax
import jax.numpy as jnp
from jax import lax
from jax.experimental import pallas as pl
from jax.experimental.pallas import tpu as pltpu
from jax.experimental.pallas import tpu_sc as plsc

_N_NODES = 100000
_N_EDGES = 3200000

N_TILES = 32
B = 1024
BLK = 128
N_BLK = B // BLK
N_CHUNKS = _N_EDGES // B
BASE_CHUNKS = N_CHUNKS // N_TILES
EXTRA = N_CHUNKS - BASE_CHUNKS * N_TILES

_LANES = 16


def _rsqrt_newton(x):
    i = lax.bitcast_convert_type(x, jnp.int32)
    i = jnp.int32(0x5F3759DF) - lax.shift_right_arithmetic(i, jnp.int32(1))
    y = lax.bitcast_convert_type(i, jnp.float32)
    for _ in range(2):
        y = y * (1.5 - 0.5 * x * y * y)
    return y


def _body(pos_hbm, bond_hbm, pbc_hbm, lat_hbm, out_hbm,
          idx_v, gi_v, gj_v, po_v, out_v, lat_v, gsem, osem, isem):
    wid = lax.axis_index("s") * 2 + lax.axis_index("c")
    pltpu.sync_copy(lat_hbm, lat_v)
    lane0 = lax.iota(jnp.int32, _LANES)
    lat = [lat_v[k] for k in range(9)]
    nchunks = jnp.int32(BASE_CHUNKS) + jnp.where(wid < EXTRA, 1, 0).astype(jnp.int32)

    def _gather_descs(p):
        descs = []
        for b in range(N_BLK):
            descs.append(pltpu.make_async_copy(
                pos_hbm.at[idx_v.at[p, b, 0]],
                gi_v.at[p].at[pl.ds(b * BLK, BLK)], gsem))
            descs.append(pltpu.make_async_copy(
                pos_hbm.at[idx_v.at[p, b, 1]],
                gj_v.at[p].at[pl.ds(b * BLK, BLK)], gsem))
        return descs

    def _pbc_descs(cid, p):
        return [
            pltpu.make_async_copy(
                pbc_hbm.at[pl.ds(cid * N_BLK, N_BLK)], po_v.at[p], gsem),
        ]

    def idx_desc(cid, p):
        return pltpu.make_async_copy(
            bond_hbm.at[pl.ds(cid * N_BLK, N_BLK)], idx_v.at[p], isem)

    def load_and_start(cid, p):
        pltpu.sync_copy(bond_hbm.at[pl.ds(cid * N_BLK, N_BLK)], idx_v.at[p])
        for d in _gather_descs(p) + _pbc_descs(cid, p):
            d.start()

    def start_after_idx(cid, p):
        idx_desc(cid, p).wait()
        for d in _gather_descs(p) + _pbc_descs(cid, p):
            d.start()

    def wait_chunk(cid, p):
        for d in _gather_descs(p) + _pbc_descs(cid, p):
            d.wait()

    def out_desc(cid, p):
        return pltpu.make_async_copy(
            out_v.at[p], out_hbm.at[pl.ds(cid * B, B)], osem)

    def compute(cid, p):
        gi = gi_v.at[p]
        gj = gj_v.at[p]
        po = po_v.at[p]
        ov = out_v.at[p]

        def step(s, c2):
            ids = lane0 + s * _LANES
            c0 = jnp.full((_LANES,), 0, jnp.int32)
            c1 = jnp.full((_LANES,), 1, jnp.int32)
            c2c = jnp.full((_LANES,), 2, jnp.int32)
            xi = plsc.load_gather(gi, [ids, c0])
            yi = plsc.load_gather(gi, [ids, c1])
            zi = plsc.load_gather(gi, [ids, c2c])
            xj = plsc.load_gather(gj, [ids, c0])
            yj = plsc.load_gather(gj, [ids, c1])
            zj = plsc.load_gather(gj, [ids, c2c])
            blk = s // 8
            off = pl.ds((s % 8) * _LANES, _LANES)
            ox = po[blk, 0, off]
            oy = po[blk, 1, off]
            oz = po[blk, 2, off]
            sl = pl.ds(s * _LANES, _LANES)
            dx = xj - xi + (ox * lat[0] + oy * lat[3] + oz * lat[6])
            dy = yj - yi + (ox * lat[1] + oy * lat[4] + oz * lat[7])
            dz = zj - zi + (ox * lat[2] + oy * lat[5] + oz * lat[8])
            ss = dx * dx + dy * dy + dz * dz
            ss = jnp.maximum(ss, jnp.float32(1e-30))
            ov[sl] = ss * _rsqrt_newton(ss)
            return c2

        lax.fori_loop(0, B // _LANES, step, jnp.int32(0), unroll=2)
        out_desc(cid, p).start()

    load_and_start(wid, 0)
    npairs = (nchunks + 1) // 2

    def pair_body(k, carry):
        c_a = wid + (2 * k) * N_TILES
        c_b = c_a + N_TILES
        has_b = (2 * k + 1) < nchunks

        @pl.when(has_b)
        def _():
            idx_desc(c_b, 1).start()

        wait_chunk(c_a, 0)

        @pl.when(has_b)
        def _():
            start_after_idx(c_b, 1)

        @pl.when(k > 0)
        def _():
            out_desc(c_a, 0).wait()

        compute(c_a, 0)

        @pl.when(has_b)
        def _():
            has_n = (2 * k + 2) < nchunks
            c_n = wid + (2 * k + 2) * N_TILES

            @pl.when(has_n)
            def _():
                idx_desc(c_n, 0).start()

            wait_chunk(c_b, 1)

            @pl.when(has_n)
            def _():
                start_after_idx(c_n, 0)

            @pl.when(k > 0)
            def _():
                out_desc(c_b, 1).wait()

            compute(c_b, 1)

        return carry

    lax.fori_loop(0, npairs, pair_body, jnp.int32(0))
    out_desc(wid, 0).wait()
    out_desc(wid, 1).wait()


_sc_call = pl.kernel(
    _body,
    out_type=jax.ShapeDtypeStruct((_N_EDGES,), jnp.float32),
    mesh=plsc.VectorSubcoreMesh(core_axis_name="c", subcore_axis_name="s",
                                num_cores=2, num_subcores=16),
    scratch_types=[
        pltpu.VMEM((2, N_BLK, 2, BLK), jnp.int32),
        pltpu.VMEM((2, B, 8), jnp.float32),
        pltpu.VMEM((2, B, 8), jnp.float32),
        pltpu.VMEM((2, N_BLK, 4, BLK), jnp.float32),
        pltpu.VMEM((2, B), jnp.float32),
        pltpu.VMEM((9, _LANES), jnp.float32),
        pltpu.SemaphoreType.DMA,
        pltpu.SemaphoreType.DMA,
        pltpu.SemaphoreType.DMA,
    ],
    compiler_params=pltpu.CompilerParams(
        needs_layout_passes=False, use_tc_tiling_on_sc=False),
)


def kernel(atom_positions, bond_atom_indices, pbc_offsets, lattices):
    pos8 = jnp.pad(atom_positions, ((0, 0), (0, 5)))
    bond_blocks = jnp.transpose(
        bond_atom_indices.reshape(_N_EDGES // BLK, BLK, 2), (0, 2, 1))
    pbc_soa = jnp.transpose(
        jnp.pad(pbc_offsets, ((0, 0), (0, 1))).reshape(
            _N_EDGES // BLK, BLK, 4), (0, 2, 1))
    lat_b = jnp.broadcast_to(lattices.reshape(9, 1), (9, 16))
    return _sc_call(pos8, bond_blocks, pbc_soa, lat_b)
